# baseline (speedup 1.0000x reference)
import functools

import jax
import jax.numpy as jnp
from jax.experimental import pallas as pl
from jax.experimental.pallas import tpu as pltpu

N_EXPERTS = 64
TOP_K = 8
TILE = 1024
H_CHUNKS = 2


def _gate_kernel(hs_ref, w_ref, idx_ref, wgt_ref, acc_ref):
    c = pl.program_id(1)
    part = jax.lax.dot_general(
        w_ref[...], hs_ref[...], (((1,), (1,)), ((), ())),
        preferred_element_type=jnp.float32,
    )

    @pl.when(c == 0)
    def _():
        acc_ref[...] = part

    @pl.when(c == H_CHUNKS - 1)
    def _():
        lt = acc_ref[...] + part

        iota = jax.lax.broadcasted_iota(jnp.int32, lt.shape, 0)
        cur = lt
        vals = []
        idxs = []
        for _ in range(TOP_K):
            m = jnp.max(cur, axis=0, keepdims=True)
            i = jnp.min(
                jnp.where(cur == m, iota, N_EXPERTS), axis=0, keepdims=True
            )
            vals.append(m)
            idxs.append(i)
            cur = jnp.where(iota == i, -jnp.inf, cur)
        v = jnp.concatenate(vals, axis=0)
        ii = jnp.concatenate(idxs, axis=0)

        e = jnp.exp(v - v[0:1])
        sm = e / jnp.sum(e, axis=0, keepdims=True)
        sm = sm / (jnp.sum(sm, axis=0, keepdims=True) + 1e-20)

        idx_ref[...] = ii.T
        wgt_ref[...] = sm.T


@functools.partial(jax.jit, static_argnames=())
def kernel(hidden_states, weight):
    bsz, seq_len, h = hidden_states.shape
    rows = bsz * seq_len
    hs = hidden_states.reshape(rows, h)
    hc = h // H_CHUNKS
    grid = (rows // TILE, H_CHUNKS)
    idx, wgt = pl.pallas_call(
        _gate_kernel,
        grid=grid,
        in_specs=[
            pl.BlockSpec((TILE, hc), lambda r, c: (r, c)),
            pl.BlockSpec((N_EXPERTS, hc), lambda r, c: (0, c)),
        ],
        out_specs=[
            pl.BlockSpec((TILE, TOP_K), lambda r, c: (r, 0)),
            pl.BlockSpec((TILE, TOP_K), lambda r, c: (r, 0)),
        ],
        out_shape=[
            jax.ShapeDtypeStruct((rows, TOP_K), jnp.int32),
            jax.ShapeDtypeStruct((rows, TOP_K), jnp.float32),
        ],
        scratch_shapes=[pltpu.VMEM((N_EXPERTS, TILE), jnp.float32)],
        compiler_params=pltpu.CompilerParams(
            dimension_semantics=("parallel", "arbitrary"),
        ),
    )(hs, weight)
    return idx, wgt


# speedup vs baseline: 1.1375x; 1.1375x over previous
import functools

import jax
import jax.numpy as jnp
from jax.experimental import pallas as pl
from jax.experimental.pallas import tpu as pltpu

N_EXPERTS = 64
TOP_K = 8
TILE = 1024


def _gate_kernel(hs_ref, w_ref, idx_ref, wgt_ref):
    hs = hs_ref[...]
    w = w_ref[...]
    lt = jax.lax.dot_general(
        w, hs, (((1,), (1,)), ((), ())), preferred_element_type=jnp.float32
    )

    iota = jax.lax.broadcasted_iota(jnp.int32, lt.shape, 0)
    cur = lt
    vals = []
    idxs = []
    for _ in range(TOP_K):
        m = jnp.max(cur, axis=0, keepdims=True)
        i = jnp.min(jnp.where(cur == m, iota, N_EXPERTS), axis=0, keepdims=True)
        vals.append(m)
        idxs.append(i)
        cur = jnp.where(iota == i, -jnp.inf, cur)
    v = jnp.concatenate(vals, axis=0)
    ii = jnp.concatenate(idxs, axis=0)

    e = jnp.exp(v - v[0:1])
    sm = e / jnp.sum(e, axis=0, keepdims=True)
    sm = sm / (jnp.sum(sm, axis=0, keepdims=True) + 1e-20)

    idx_ref[...] = ii.T
    wgt_ref[...] = sm.T


@functools.partial(jax.jit, static_argnames=())
def kernel(hidden_states, weight):
    bsz, seq_len, h = hidden_states.shape
    rows = bsz * seq_len
    hs = hidden_states.reshape(rows, h)
    grid = (rows // TILE,)
    idx, wgt = pl.pallas_call(
        _gate_kernel,
        grid=grid,
        in_specs=[
            pl.BlockSpec((TILE, h), lambda r: (r, 0)),
            pl.BlockSpec((N_EXPERTS, h), lambda r: (0, 0)),
        ],
        out_specs=[
            pl.BlockSpec((TILE, TOP_K), lambda r: (r, 0)),
            pl.BlockSpec((TILE, TOP_K), lambda r: (r, 0)),
        ],
        out_shape=[
            jax.ShapeDtypeStruct((rows, TOP_K), jnp.int32),
            jax.ShapeDtypeStruct((rows, TOP_K), jnp.float32),
        ],
        compiler_params=pltpu.CompilerParams(
            dimension_semantics=("parallel",),
        ),
    )(hs, weight)
    return idx, wgt
